# baseline (speedup 1.0000x reference)
import jax
import jax.numpy as jnp
import numpy as np
from jax import lax
from jax.experimental import pallas as pl
from jax.experimental.pallas import tpu as pltpu
from jax.experimental.pallas import tpu_sc as plsc

N = 10000
E = 320000
EPAD = 327680
D = 128
NC = 2
NS = 16
L = 16
NP = 4
F = D // NP
EPT = EPAD // NS
C = 128
NCHUNK = EPT // C
NPAD = 10240
NPT = NPAD // NS
MM_BLK = 2000
MM_GRID = (2 * N) // MM_BLK


def _mm_body(s_ref, w_ref, lw_ref, b_ref, a_ref, lb_ref, x_ref, aux_ref):
    x_ref[0] = jnp.dot(s_ref[...], w_ref[0],
                       preferred_element_type=jnp.float32)

    @pl.when(jnp.logical_and(pl.program_id(0) == 0, pl.program_id(1) == 0))
    def _():
        aux_ref[0:1, :] = b_ref[...]
        aux_ref[1:2, :] = jnp.sum(lw_ref[...], axis=0, keepdims=True)
        cc = jnp.sum(lb_ref[...])
        ii = lax.broadcasted_iota(jnp.int32, (1, D), 1)
        a = a_ref[0, 0]
        aux_ref[2:3, :] = jnp.where(ii == 0, a, jnp.where(ii == 1, cc, 0.0))
        aux_ref[3:8, :] = jnp.zeros((5, D), jnp.float32)


_mm = pl.pallas_call(
    _mm_body,
    grid=(MM_GRID, NP),
    in_specs=[
        pl.BlockSpec((MM_BLK, D), lambda i, j: (i, 0)),
        pl.BlockSpec((1, D, F), lambda i, j: (j, 0, 0)),
        pl.BlockSpec((D, D), lambda i, j: (0, 0)),
        pl.BlockSpec((1, D), lambda i, j: (0, 0)),
        pl.BlockSpec((1, D), lambda i, j: (0, 0)),
        pl.BlockSpec((1, D), lambda i, j: (0, 0)),
    ],
    out_specs=[
        pl.BlockSpec((1, MM_BLK, F), lambda i, j: (j, i, 0)),
        pl.BlockSpec((8, D), lambda i, j: (0, 0)),
    ],
    out_shape=[
        jax.ShapeDtypeStruct((NP, 2 * N, F), jnp.float32),
        jax.ShapeDtypeStruct((8, D), jnp.float32),
    ],
)


NB = 4
BE = NB * C
NBATCH = EPT // BE
NDB = NBATCH // 2


def _sc_body(xf_hbm, src_hbm, dst_hbm, w_hbm, aux_hbm, out_hbm,
             sb_v, db_v, wb_v, gidx_v, w2_v, dst2_v, rows2_v, zer_v,
             slab_v, pacc_v, aux_v, out_v, agg_sh,
             isem, gsemA, gsemB, ssemA, ssemB):
    c = lax.axis_index("c")
    s = lax.axis_index("s")
    gsems = [gsemA, gsemB]
    ssems = [ssemA, ssemB]

    pltpu.sync_copy(aux_hbm, aux_v)

    zero16 = jnp.zeros((L,), jnp.float32)
    iota16 = lax.broadcasted_iota(jnp.int32, (L,), 0)

    def zz(i, carry):
        for j in range(F // L):
            zer_v[i, pl.ds(j * L, L)] = zero16
        pacc_v[i, pl.ds(0, L)] = zero16
        return carry

    lax.fori_loop(0, NPT, zz, 0)

    pvec = aux_v[2, pl.ds(0, L)]
    a = pvec[0]
    cc = pvec[1]

    ebase = s * EPT
    nb = s * NPT

    def fire_idx(b, buf):
        base = ebase + b * BE
        pltpu.async_copy(src_hbm.at[pl.ds(base, BE)], sb_v.at[buf], isem)
        pltpu.async_copy(dst_hbm.at[pl.ds(base, BE)], db_v.at[buf], isem)
        pltpu.async_copy(w_hbm.at[pl.ds(base, BE)], wb_v.at[buf], isem)

    def drain_idx(buf):
        pltpu.make_async_copy(src_hbm.at[pl.ds(0, BE)],
                              sb_v.at[buf], isem).wait()
        pltpu.make_async_copy(dst_hbm.at[pl.ds(0, BE)],
                              db_v.at[buf], isem).wait()
        pltpu.make_async_copy(w_hbm.at[pl.ds(0, BE)],
                              wb_v.at[buf], isem).wait()

    def prep(buf, xoff):
        for k in range(NB):
            for j in range(C // L):
                sl16 = pl.ds(k * C + j * L, L)
                gidx_v[buf, k, pl.ds(j * L, L)] = sb_v[buf, sl16] + xoff
                dst2_v[buf, k, pl.ds(j * L, L)] = db_v[buf, sl16]
        for j in range(BE // L):
            sl = pl.ds(j * L, L)
            w2_v[buf, sl] = wb_v[buf, sl]

    def fire_gathers(buf):
        for k in range(NB):
            pltpu.async_copy(
                xf_hbm.at[gidx_v.at[buf, k]],
                rows2_v.at[buf, k], gsems[buf])

    def wait_gather(buf, k):
        pltpu.make_async_copy(
            xf_hbm.at[gidx_v.at[buf, k]],
            rows2_v.at[buf, k], gsems[buf]).wait()

    def drain_scatters(buf):
        for k in range(NB):
            pltpu.make_async_copy(
                xf_hbm.at[pl.ds(0, C)], rows2_v.at[buf, k],
                ssems[buf]).wait()

    def process(buf):
        for k in range(NB):
            wait_gather(buf, k)

            def mul(g, mcarry, _k=k, _buf=buf):
                w16 = w2_v[_buf, pl.ds(_k * C + g * L, L)]
                for kk in range(L):
                    ws = w16.at[jnp.full((L,), kk, jnp.int32)].get(
                        mode="promise_in_bounds")
                    e = g * L + kk
                    for j in range(F // L):
                        sl = pl.ds(j * L, L)
                        rows2_v[_buf, _k, e, sl] = \
                            rows2_v[_buf, _k, e, sl] * ws
                return mcarry

            lax.fori_loop(0, C // L, mul, 0)
            pltpu.async_copy(rows2_v.at[buf, k],
                             agg_sh.at[dst2_v.at[buf, k]],
                             ssems[buf], add=True)

    def handle(b, buf, xoff):
        nxt = 1 - buf

        @pl.when(b >= 1)
        def _():
            drain_scatters(nxt)

        @pl.when(b + 1 < NBATCH)
        def _():
            drain_idx(nxt)

        @pl.when(b + 1 < NBATCH)
        def _():
            prep(nxt, xoff)

        @pl.when(b + 2 < NBATCH)
        def _():
            fire_idx(b + 2, buf)

        @pl.when(b + 1 < NBATCH)
        def _():
            fire_gathers(nxt)

        process(buf)

    def phase(p, carry):
        pltpu.sync_copy(zer_v, agg_sh.at[pl.ds(nb, NPT)])
        plsc.subcore_barrier()

        xoff = p * (2 * N) + c * N
        fire_idx(0, 0)
        drain_idx(0)
        prep(0, xoff)
        fire_idx(1, 1)
        fire_gathers(0)

        def dbatch(t, dcarry):
            handle(2 * t, 0, xoff)
            handle(2 * t + 1, 1, xoff)
            return dcarry

        lax.fori_loop(0, NDB, dbatch, 0)
        drain_scatters(1)
        plsc.subcore_barrier()

        pltpu.sync_copy(agg_sh.at[pl.ds(nb, NPT)], slab_v)
        plsc.subcore_barrier()

        b_ph = [aux_v[0, pl.ds(p * F + j * L, L)] for j in range(F // L)]
        v_ph = [aux_v[1, pl.ds(p * F + j * L, L)] for j in range(F // L)]

        def fold(n, fcarry):
            acc = pacc_v[n, pl.ds(0, L)]
            for j in range(F // L):
                y = slab_v[n, pl.ds(j * L, L)] + b_ph[j]
                q = jnp.where(y >= 0.0, y, y * a)
                acc = acc + q * v_ph[j]
            pacc_v[n, pl.ds(0, L)] = acc
            return fcarry

        lax.fori_loop(0, NPT, fold, 0)
        return carry

    lax.fori_loop(0, NP, phase, 0)

    def ro(g, carry):
        outvec = jnp.zeros((L,), jnp.float32)
        for k in range(L):
            acc = pacc_v[g * L + k, pl.ds(0, L)]
            sval = cc
            for i in range(L):
                sval = sval + acc[i]
            outvec = jnp.where(iota16 == k, sval, outvec)
        out_v[pl.ds(g * L, L)] = outvec
        return carry

    lax.fori_loop(0, NPT // L, ro, 0)
    pltpu.sync_copy(out_v, out_hbm.at[c, s])


_sc_spmm = pl.kernel(
    _sc_body,
    out_type=jax.ShapeDtypeStruct((NC, NS, NPT), jnp.float32),
    mesh=plsc.VectorSubcoreMesh(core_axis_name="c", subcore_axis_name="s",
                                num_cores=NC, num_subcores=NS),
    compiler_params=pltpu.CompilerParams(use_tc_tiling_on_sc=False),
    scratch_types=[
        pltpu.VMEM((2, BE), jnp.int32),
        pltpu.VMEM((2, BE), jnp.int32),
        pltpu.VMEM((2, BE), jnp.float32),
        pltpu.VMEM((2, NB, C), jnp.int32),
        pltpu.VMEM((2, BE), jnp.float32),
        pltpu.VMEM((2, NB, C), jnp.int32),
        pltpu.VMEM((2, NB, C, F), jnp.float32),
        pltpu.VMEM((NPT, F), jnp.float32),
        pltpu.VMEM((NPT, F), jnp.float32),
        pltpu.VMEM((NPT, L), jnp.float32),
        pltpu.VMEM((8, D), jnp.float32),
        pltpu.VMEM((NPT,), jnp.float32),
        pltpu.VMEM_SHARED((NPAD, F), jnp.float32),
        pltpu.SemaphoreType.DMA,
        pltpu.SemaphoreType.DMA,
        pltpu.SemaphoreType.DMA,
        pltpu.SemaphoreType.DMA,
        pltpu.SemaphoreType.DMA,
    ],
)


def kernel(seq1, seq2, edge_index, edge_weight, sparse, W_gcn, b_gcn,
           prelu_a, lin_W, lin_b):
    del sparse
    S = jnp.concatenate([seq1[0], seq2[0]], axis=0)
    b_row = b_gcn.reshape(1, D)
    a_row = jnp.zeros((1, D), jnp.float32).at[0, 0].set(prelu_a[0])
    lb_row = lin_b.reshape(1, D)
    W4 = W_gcn.reshape(D, NP, F).transpose(1, 0, 2)
    x4, aux = _mm(S, W4, lin_W, b_row, a_row, lb_row)
    xf = x4.reshape(NP * 2 * N, F)
    pad = EPAD - E
    spread = (jnp.arange(pad, dtype=jnp.int32) % (NPAD - N)) + N
    src_p = jnp.concatenate([edge_index[1], jnp.zeros((pad,), jnp.int32)])
    dst_p = jnp.concatenate([edge_index[0], spread])
    w_p = jnp.concatenate([edge_weight, jnp.zeros((pad,), jnp.float32)])
    out3 = _sc_spmm(xf, src_p, dst_p, w_p, aux)
    return out3.reshape(NC, NPAD)[:, :N].reshape(1, 2 * N)


# speedup vs baseline: 1.8902x; 1.8902x over previous
import jax
import jax.numpy as jnp
import numpy as np
from jax import lax
from jax.experimental import pallas as pl
from jax.experimental.pallas import tpu as pltpu
from jax.experimental.pallas import tpu_sc as plsc

N = 10000
E = 320000
D = 128
NC = 2
NS = 16
L = 16
NP = 4
F = D // NP
EPT = E // NS
C = 80
NCHUNK = EPT // C
NPAD = 10240
NPT = NPAD // NS
MM_BLK = 2000
MM_GRID = (2 * N) // MM_BLK


def _mm_body(s_ref, w_ref, lw_ref, b_ref, a_ref, lb_ref, x_ref, aux_ref):
    x_ref[0] = jnp.dot(s_ref[...], w_ref[0],
                       preferred_element_type=jnp.float32)

    @pl.when(jnp.logical_and(pl.program_id(0) == 0, pl.program_id(1) == 0))
    def _():
        aux_ref[0:1, :] = b_ref[...]
        aux_ref[1:2, :] = jnp.sum(lw_ref[...], axis=0, keepdims=True)
        cc = jnp.sum(lb_ref[...])
        ii = lax.broadcasted_iota(jnp.int32, (1, D), 1)
        a = a_ref[0, 0]
        aux_ref[2:3, :] = jnp.where(ii == 0, a, jnp.where(ii == 1, cc, 0.0))
        aux_ref[3:8, :] = jnp.zeros((5, D), jnp.float32)


_mm = pl.pallas_call(
    _mm_body,
    grid=(MM_GRID, NP),
    in_specs=[
        pl.BlockSpec((MM_BLK, D), lambda i, j: (i, 0)),
        pl.BlockSpec((1, D, F), lambda i, j: (j, 0, 0)),
        pl.BlockSpec((D, D), lambda i, j: (0, 0)),
        pl.BlockSpec((1, D), lambda i, j: (0, 0)),
        pl.BlockSpec((1, D), lambda i, j: (0, 0)),
        pl.BlockSpec((1, D), lambda i, j: (0, 0)),
    ],
    out_specs=[
        pl.BlockSpec((1, MM_BLK, F), lambda i, j: (j, i, 0)),
        pl.BlockSpec((8, D), lambda i, j: (0, 0)),
    ],
    out_shape=[
        jax.ShapeDtypeStruct((NP, 2 * N, F), jnp.float32),
        jax.ShapeDtypeStruct((8, D), jnp.float32),
    ],
)


NB = 5
BE = NB * C
NBATCH = EPT // BE
NDB = NBATCH // 2


def _sc_body(xf_hbm, src_hbm, dst_hbm, w_hbm, aux_hbm, out_hbm,
             sb_v, db_v, wb_v, gidx_v, w2_v, dst2_v, rows2_v, zer_v,
             slab_v, pacc_v, aux_v, out_v, agg_sh,
             isem, gsemA, gsemB, ssemA, ssemB):
    c = lax.axis_index("c")
    s = lax.axis_index("s")
    gsems = [gsemA, gsemB]
    ssems = [ssemA, ssemB]

    pltpu.sync_copy(aux_hbm, aux_v)

    zero16 = jnp.zeros((L,), jnp.float32)
    iota16 = lax.broadcasted_iota(jnp.int32, (L,), 0)

    def zz(i, carry):
        for j in range(F // L):
            zer_v[i, pl.ds(j * L, L)] = zero16
        pacc_v[i, pl.ds(0, L)] = zero16
        return carry

    lax.fori_loop(0, NPT, zz, 0)

    pvec = aux_v[2, pl.ds(0, L)]
    a = pvec[0]
    cc = pvec[1]

    ebase = s * EPT
    nb = s * NPT

    def fire_idx(b, buf):
        base = ebase + b * BE
        pltpu.async_copy(src_hbm.at[pl.ds(base, BE)], sb_v.at[buf], isem)
        pltpu.async_copy(dst_hbm.at[pl.ds(base, BE)], db_v.at[buf], isem)
        pltpu.async_copy(w_hbm.at[pl.ds(base, BE)], wb_v.at[buf], isem)

    def drain_idx(buf):
        pltpu.make_async_copy(src_hbm.at[pl.ds(0, BE)],
                              sb_v.at[buf], isem).wait()
        pltpu.make_async_copy(dst_hbm.at[pl.ds(0, BE)],
                              db_v.at[buf], isem).wait()
        pltpu.make_async_copy(w_hbm.at[pl.ds(0, BE)],
                              wb_v.at[buf], isem).wait()

    def prep(buf, xoff):
        for k in range(NB):
            for j in range(C // L):
                sl16 = pl.ds(k * C + j * L, L)
                gidx_v[buf, k, pl.ds(j * L, L)] = sb_v[buf, sl16] + xoff
                dst2_v[buf, k, pl.ds(j * L, L)] = db_v[buf, sl16]
        for j in range(BE // L):
            sl = pl.ds(j * L, L)
            w2_v[buf, sl] = wb_v[buf, sl]

    def fire_gathers(buf):
        for k in range(NB):
            pltpu.async_copy(
                xf_hbm.at[gidx_v.at[buf, k]],
                rows2_v.at[buf, k], gsems[buf])

    def wait_gather(buf, k):
        pltpu.make_async_copy(
            xf_hbm.at[gidx_v.at[buf, k]],
            rows2_v.at[buf, k], gsems[buf]).wait()

    def drain_scatters(buf):
        for k in range(NB):
            pltpu.make_async_copy(
                xf_hbm.at[pl.ds(0, C)], rows2_v.at[buf, k],
                ssems[buf]).wait()

    def process(buf):
        for k in range(NB):
            wait_gather(buf, k)

            def mul(g, mcarry, _k=k, _buf=buf):
                w16 = w2_v[_buf, pl.ds(_k * C + g * L, L)]
                for kk in range(L):
                    ws = w16.at[jnp.full((L,), kk, jnp.int32)].get(
                        mode="promise_in_bounds")
                    e = g * L + kk
                    for j in range(F // L):
                        sl = pl.ds(j * L, L)
                        rows2_v[_buf, _k, e, sl] = \
                            rows2_v[_buf, _k, e, sl] * ws
                return mcarry

            lax.fori_loop(0, C // L, mul, 0)
            pltpu.async_copy(rows2_v.at[buf, k],
                             agg_sh.at[dst2_v.at[buf, k]],
                             ssems[buf], add=True)

    def handle(b, buf, xoff):
        nxt = 1 - buf

        @pl.when(b >= 1)
        def _():
            drain_scatters(nxt)

        @pl.when(b + 1 < NBATCH)
        def _():
            drain_idx(nxt)

        @pl.when(b + 1 < NBATCH)
        def _():
            prep(nxt, xoff)

        @pl.when(b + 2 < NBATCH)
        def _():
            fire_idx(b + 2, buf)

        @pl.when(b + 1 < NBATCH)
        def _():
            fire_gathers(nxt)

        process(buf)

    def phase(p, carry):
        pltpu.sync_copy(zer_v, agg_sh.at[pl.ds(nb, NPT)])
        plsc.subcore_barrier()

        xoff = p * (2 * N) + c * N
        fire_idx(0, 0)
        drain_idx(0)
        prep(0, xoff)
        fire_idx(1, 1)
        fire_gathers(0)

        def dbatch(t, dcarry):
            handle(2 * t, 0, xoff)
            handle(2 * t + 1, 1, xoff)
            return dcarry

        lax.fori_loop(0, NDB, dbatch, 0)
        drain_scatters(1)
        plsc.subcore_barrier()

        pltpu.sync_copy(agg_sh.at[pl.ds(nb, NPT)], slab_v)
        plsc.subcore_barrier()

        b_ph = [aux_v[0, pl.ds(p * F + j * L, L)] for j in range(F // L)]
        v_ph = [aux_v[1, pl.ds(p * F + j * L, L)] for j in range(F // L)]

        def fold(n, fcarry):
            acc = pacc_v[n, pl.ds(0, L)]
            for j in range(F // L):
                y = slab_v[n, pl.ds(j * L, L)] + b_ph[j]
                q = jnp.where(y >= 0.0, y, y * a)
                acc = acc + q * v_ph[j]
            pacc_v[n, pl.ds(0, L)] = acc
            return fcarry

        lax.fori_loop(0, NPT, fold, 0)
        return carry

    lax.fori_loop(0, NP, phase, 0)

    def ro(g, carry):
        outvec = jnp.zeros((L,), jnp.float32)
        for k in range(L):
            acc = pacc_v[g * L + k, pl.ds(0, L)]
            sval = cc
            for i in range(L):
                sval = sval + acc[i]
            outvec = jnp.where(iota16 == k, sval, outvec)
        out_v[pl.ds(g * L, L)] = outvec
        return carry

    lax.fori_loop(0, NPT // L, ro, 0)
    pltpu.sync_copy(out_v, out_hbm.at[c, s])


_sc_spmm = pl.kernel(
    _sc_body,
    out_type=jax.ShapeDtypeStruct((NC, NS, NPT), jnp.float32),
    mesh=plsc.VectorSubcoreMesh(core_axis_name="c", subcore_axis_name="s",
                                num_cores=NC, num_subcores=NS),
    compiler_params=pltpu.CompilerParams(use_tc_tiling_on_sc=False),
    scratch_types=[
        pltpu.VMEM((2, BE), jnp.int32),
        pltpu.VMEM((2, BE), jnp.int32),
        pltpu.VMEM((2, BE), jnp.float32),
        pltpu.VMEM((2, NB, C), jnp.int32),
        pltpu.VMEM((2, BE), jnp.float32),
        pltpu.VMEM((2, NB, C), jnp.int32),
        pltpu.VMEM((2, NB, C, F), jnp.float32),
        pltpu.VMEM((NPT, F), jnp.float32),
        pltpu.VMEM((NPT, F), jnp.float32),
        pltpu.VMEM((NPT, L), jnp.float32),
        pltpu.VMEM((8, D), jnp.float32),
        pltpu.VMEM((NPT,), jnp.float32),
        pltpu.VMEM_SHARED((NPAD, F), jnp.float32),
        pltpu.SemaphoreType.DMA,
        pltpu.SemaphoreType.DMA,
        pltpu.SemaphoreType.DMA,
        pltpu.SemaphoreType.DMA,
        pltpu.SemaphoreType.DMA,
    ],
)


def kernel(seq1, seq2, edge_index, edge_weight, sparse, W_gcn, b_gcn,
           prelu_a, lin_W, lin_b):
    del sparse
    S = jnp.concatenate([seq1[0], seq2[0]], axis=0)
    b_row = b_gcn.reshape(1, D)
    a_row = jnp.zeros((1, D), jnp.float32).at[0, 0].set(prelu_a[0])
    lb_row = lin_b.reshape(1, D)
    W4 = W_gcn.reshape(D, NP, F).transpose(1, 0, 2)
    x4, aux = _mm(S, W4, lin_W, b_row, a_row, lb_row)
    xf = x4.reshape(NP * 2 * N, F)
    out3 = _sc_spmm(xf, edge_index[1], edge_index[0], edge_weight, aux)
    return out3.reshape(NC, NPAD)[:, :N].reshape(1, 2 * N)
